# Initial kernel scaffold; baseline (speedup 1.0000x reference)
import jax
import jax.numpy as jnp
from jax.experimental import pallas as pl


def kernel(edge_index, emb_weight):
    raise NotImplementedError("write your pallas kernel here")


# speedup vs baseline: 8.1814x; 8.1814x over previous
import functools

import jax
import jax.numpy as jnp
from jax import lax
from jax.experimental import pallas as pl
from jax.experimental.pallas import tpu as pltpu
from jax.experimental.pallas import tpu_sc as plsc

N = 10000
D = 128
DH = 64
E = 320000
LAYERS = 3
NC, NS = 2, 16
NW = NC * NS
NPAD = 10240
C = 128
EW = 10112
NCH = EW // C
EPAD = NW * EW
RPS = NPAD // NS


def _deg_body(dst_hbm, deg_out, dst_v, ones_v, zvec, acc_sh, zsem, ssem):
    c = lax.axis_index("c")
    s = lax.axis_index("s")
    w = s * NC + c
    for i in range(RPS // 16):
        zvec[pl.ds(i * 16, 16)] = jnp.zeros((16,), jnp.float32)
    for i in range(C // 16):
        ones_v[pl.ds(i * 16, 16)] = jnp.ones((16,), jnp.float32)
    pltpu.sync_copy(dst_hbm.at[w], dst_v)
    pltpu.sync_copy(zvec, acc_sh.at[pl.ds(s * RPS, RPS)])
    plsc.subcore_barrier()
    K = 8
    for g in range(0, NCH, K):
        n = min(K, NCH - g)
        cps = [
            pltpu.async_copy(ones_v, acc_sh.at[dst_v.at[g + t]], ssem, add=True)
            for t in range(n)
        ]
        for cp in cps:
            cp.wait()
    plsc.subcore_barrier()
    pltpu.sync_copy(acc_sh.at[pl.ds(s * RPS, RPS)],
                    deg_out.at[c, pl.ds(s * RPS, RPS)])


_deg_kernel = functools.partial(
    pl.kernel,
    out_type=jax.ShapeDtypeStruct((NC, NPAD), jnp.float32),
    mesh=plsc.VectorSubcoreMesh(core_axis_name="c", subcore_axis_name="s"),
    scratch_types=[
        pltpu.VMEM((NCH, C), jnp.int32),
        pltpu.VMEM((C,), jnp.float32),
        pltpu.VMEM((RPS,), jnp.float32),
        pltpu.VMEM_SHARED((NPAD,), jnp.float32),
        pltpu.SemaphoreType.DMA,
        pltpu.SemaphoreType.DMA,
    ],
)(_deg_body)


def _edge_body(src_hbm, dst_hbm, xp_hbm, part_out,
               src_v, dst_v, rows0, rows1, zrow, acc_sh, gsem0, gsem1):
    c = lax.axis_index("c")
    s = lax.axis_index("s")
    w = s * NC + c
    for r in range(16):
        for q in range(DH // 16):
            zrow[r, pl.ds(q * 16, 16)] = jnp.zeros((16,), jnp.float32)
    pltpu.sync_copy(src_hbm.at[w], src_v)
    pltpu.sync_copy(dst_hbm.at[w], dst_v)
    base = s * RPS
    zcps = [
        pltpu.async_copy(zrow, acc_sh.at[pl.ds(base + i * 16, 16)], gsem0)
        for i in range(RPS // 16)
    ]
    for cp in zcps:
        cp.wait()
    plsc.subcore_barrier()
    bufs = (rows0, rows1)
    sems = (gsem0, gsem1)
    cps = [None, None]
    cps[0] = pltpu.async_copy(xp_hbm.at[src_v.at[0]], rows0, gsem0)
    for j in range(NCH):
        b = j % 2
        if j + 1 < NCH:
            cps[1 - b] = pltpu.async_copy(
                xp_hbm.at[src_v.at[j + 1]], bufs[1 - b], sems[1 - b])
        cps[b].wait()
        pltpu.sync_copy(bufs[b], acc_sh.at[dst_v.at[j]], add=True)
    plsc.subcore_barrier()
    pltpu.sync_copy(acc_sh.at[pl.ds(base, RPS)],
                    part_out.at[c, pl.ds(base, RPS)])


_edge_kernel = functools.partial(
    pl.kernel,
    out_type=jax.ShapeDtypeStruct((NC, NPAD, DH), jnp.float32),
    mesh=plsc.VectorSubcoreMesh(core_axis_name="c", subcore_axis_name="s"),
    scratch_types=[
        pltpu.VMEM((NCH, C), jnp.int32),
        pltpu.VMEM((NCH, C), jnp.int32),
        pltpu.VMEM((C, DH), jnp.float32),
        pltpu.VMEM((C, DH), jnp.float32),
        pltpu.VMEM((16, DH), jnp.float32),
        pltpu.VMEM_SHARED((NPAD, DH), jnp.float32),
        pltpu.SemaphoreType.DMA,
        pltpu.SemaphoreType.DMA,
    ],
    compiler_params=pltpu.CompilerParams(use_tc_tiling_on_sc=False),
)(_edge_body)


BLK = 1024


def _prep_body(d0_ref, d1_ref, x_ref, a_ref, xpa_ref, xpb_ref):
    i = pl.program_id(0)
    deg = d0_ref[...] + d1_ref[...]
    rows = lax.broadcasted_iota(jnp.int32, (BLK, 1), 0) + i * BLK
    a = jnp.where((deg > 0) & (rows < N),
                  lax.rsqrt(jnp.maximum(deg, 1e-30)), 0.0)
    a_ref[...] = a
    xp = a * x_ref[...]
    xpa_ref[...] = xp[:, :DH]
    xpb_ref[...] = xp[:, DH:]


_prep_kernel = pl.pallas_call(
    _prep_body,
    grid=(NPAD // BLK,),
    in_specs=[
        pl.BlockSpec((BLK, 1), lambda i: (i, 0)),
        pl.BlockSpec((BLK, 1), lambda i: (i, 0)),
        pl.BlockSpec((BLK, D), lambda i: (i, 0)),
    ],
    out_specs=[
        pl.BlockSpec((BLK, 1), lambda i: (i, 0)),
        pl.BlockSpec((BLK, DH), lambda i: (i, 0)),
        pl.BlockSpec((BLK, DH), lambda i: (i, 0)),
    ],
    out_shape=[
        jax.ShapeDtypeStruct((NPAD, 1), jnp.float32),
        jax.ShapeDtypeStruct((NPAD, DH), jnp.float32),
        jax.ShapeDtypeStruct((NPAD, DH), jnp.float32),
    ],
)


def _comb_body(p0a_ref, p1a_ref, p0b_ref, p1b_ref, a_ref, s_ref,
               so_ref, xpa_ref, xpb_ref):
    a = a_ref[...]
    xa = a * (p0a_ref[...] + p1a_ref[...])
    xb = a * (p0b_ref[...] + p1b_ref[...])
    so_ref[...] = s_ref[...] + jnp.concatenate([xa, xb], axis=1)
    xpa_ref[...] = a * xa
    xpb_ref[...] = a * xb


_comb_kernel = pl.pallas_call(
    _comb_body,
    grid=(NPAD // BLK,),
    in_specs=[
        pl.BlockSpec((BLK, DH), lambda i: (i, 0)),
        pl.BlockSpec((BLK, DH), lambda i: (i, 0)),
        pl.BlockSpec((BLK, DH), lambda i: (i, 0)),
        pl.BlockSpec((BLK, DH), lambda i: (i, 0)),
        pl.BlockSpec((BLK, 1), lambda i: (i, 0)),
        pl.BlockSpec((BLK, D), lambda i: (i, 0)),
    ],
    out_specs=[
        pl.BlockSpec((BLK, D), lambda i: (i, 0)),
        pl.BlockSpec((BLK, DH), lambda i: (i, 0)),
        pl.BlockSpec((BLK, DH), lambda i: (i, 0)),
    ],
    out_shape=[
        jax.ShapeDtypeStruct((NPAD, D), jnp.float32),
        jax.ShapeDtypeStruct((NPAD, DH), jnp.float32),
        jax.ShapeDtypeStruct((NPAD, DH), jnp.float32),
    ],
)


def _final_body(p0a_ref, p1a_ref, p0b_ref, p1b_ref, a_ref, s_ref, o_ref):
    a = a_ref[...]
    xa = a * (p0a_ref[...] + p1a_ref[...])
    xb = a * (p0b_ref[...] + p1b_ref[...])
    x = jnp.concatenate([xa, xb], axis=1)
    o_ref[...] = (s_ref[...] + x) * jnp.float32(1.0 / (LAYERS + 1))


_final_kernel = pl.pallas_call(
    _final_body,
    grid=(NPAD // BLK,),
    in_specs=[
        pl.BlockSpec((BLK, DH), lambda i: (i, 0)),
        pl.BlockSpec((BLK, DH), lambda i: (i, 0)),
        pl.BlockSpec((BLK, DH), lambda i: (i, 0)),
        pl.BlockSpec((BLK, DH), lambda i: (i, 0)),
        pl.BlockSpec((BLK, 1), lambda i: (i, 0)),
        pl.BlockSpec((BLK, D), lambda i: (i, 0)),
    ],
    out_specs=[pl.BlockSpec((BLK, D), lambda i: (i, 0))],
    out_shape=[jax.ShapeDtypeStruct((NPAD, D), jnp.float32)],
)


def kernel(edge_index, emb_weight):
    src = edge_index[0].astype(jnp.int32)
    dst = edge_index[1].astype(jnp.int32)
    pad = jnp.full((EPAD - E,), N, jnp.int32)
    src_p = jnp.concatenate([src, pad]).reshape(NW, NCH, C)
    dst_p = jnp.concatenate([dst, pad]).reshape(NW, NCH, C)
    x0 = jnp.pad(emb_weight, ((0, NPAD - N), (0, 0)))

    deg2 = _deg_kernel(dst_p)
    a, xpa, xpb = _prep_kernel(deg2[0, :, None], deg2[1, :, None], x0)
    s = x0
    for layer in range(LAYERS):
        pa = _edge_kernel(src_p, dst_p, xpa)
        pb = _edge_kernel(src_p, dst_p, xpb)
        if layer + 1 < LAYERS:
            s, xpa, xpb = _comb_kernel(pa[0], pa[1], pb[0], pb[1], a, s)
        else:
            (out,) = _final_kernel(pa[0], pa[1], pb[0], pb[1], a, s)
    return (emb_weight, out[:N])


# speedup vs baseline: 13.8179x; 1.6889x over previous
import functools

import jax
import jax.numpy as jnp
from jax import lax
from jax.experimental import pallas as pl
from jax.experimental.pallas import tpu as pltpu
from jax.experimental.pallas import tpu_sc as plsc

N = 10000
D = 128
DH = 64
E = 320000
LAYERS = 3
NC, NS = 2, 16
NW = NC * NS
NPAD = 10240
C = 128
RPS = NPAD // NS

EW32 = 10112
NCH32 = EW32 // C
EPAD32 = NW * EW32

EW16 = 20096
NCH16 = EW16 // C
EPAD16 = NS * EW16


def _deg_body(dst_hbm, deg_out, dst_v, ones_v, zvec, acc_sh, zsem, ssem):
    c = lax.axis_index("c")
    s = lax.axis_index("s")
    w = s * NC + c
    for i in range(RPS // 16):
        zvec[pl.ds(i * 16, 16)] = jnp.zeros((16,), jnp.float32)
    for i in range(C // 16):
        ones_v[pl.ds(i * 16, 16)] = jnp.ones((16,), jnp.float32)
    pltpu.sync_copy(dst_hbm.at[w], dst_v)
    pltpu.sync_copy(zvec, acc_sh.at[pl.ds(s * RPS, RPS)])
    plsc.subcore_barrier()
    K = 8
    for g in range(0, NCH32, K):
        n = min(K, NCH32 - g)
        cps = [
            pltpu.async_copy(ones_v, acc_sh.at[dst_v.at[g + t]], ssem, add=True)
            for t in range(n)
        ]
        for cp in cps:
            cp.wait()
    plsc.subcore_barrier()
    pltpu.sync_copy(acc_sh.at[pl.ds(s * RPS, RPS)],
                    deg_out.at[c, pl.ds(s * RPS, RPS)])


_deg_kernel = functools.partial(
    pl.kernel,
    out_type=jax.ShapeDtypeStruct((NC, NPAD), jnp.float32),
    mesh=plsc.VectorSubcoreMesh(core_axis_name="c", subcore_axis_name="s"),
    scratch_types=[
        pltpu.VMEM((NCH32, C), jnp.int32),
        pltpu.VMEM((C,), jnp.float32),
        pltpu.VMEM((RPS,), jnp.float32),
        pltpu.VMEM_SHARED((NPAD,), jnp.float32),
        pltpu.SemaphoreType.DMA,
        pltpu.SemaphoreType.DMA,
    ],
)(_deg_body)


def _edge_body(src_hbm, dst_hbm, xp2_hbm, y_out,
               src_v, dst_v, rows0, rows1, zrow, acc_sh, gsem0, gsem1):
    c = lax.axis_index("c")
    s = lax.axis_index("s")
    for r in range(16):
        for q in range(DH // 16):
            zrow[r, pl.ds(q * 16, 16)] = jnp.zeros((16,), jnp.float32)
    pltpu.sync_copy(src_hbm.at[s], src_v)
    pltpu.sync_copy(dst_hbm.at[s], dst_v)
    base = s * RPS
    zcps = [
        pltpu.async_copy(zrow, acc_sh.at[pl.ds(base + i * 16, 16)], gsem0)
        for i in range(RPS // 16)
    ]
    for cp in zcps:
        cp.wait()
    plsc.subcore_barrier()
    xp_h = xp2_hbm.at[c]
    bufs = (rows0, rows1)
    sems = (gsem0, gsem1)
    cps = [None, None]
    cps[0] = pltpu.async_copy(xp_h.at[src_v.at[0]], rows0, gsem0)
    for j in range(NCH16):
        b = j % 2
        if j + 1 < NCH16:
            cps[1 - b] = pltpu.async_copy(
                xp_h.at[src_v.at[j + 1]], bufs[1 - b], sems[1 - b])
        cps[b].wait()
        pltpu.sync_copy(bufs[b], acc_sh.at[dst_v.at[j]], add=True)
    plsc.subcore_barrier()
    pltpu.sync_copy(acc_sh.at[pl.ds(base, RPS)],
                    y_out.at[c, pl.ds(base, RPS)])


_edge_kernel = functools.partial(
    pl.kernel,
    out_type=jax.ShapeDtypeStruct((NC, NPAD, DH), jnp.float32),
    mesh=plsc.VectorSubcoreMesh(core_axis_name="c", subcore_axis_name="s"),
    scratch_types=[
        pltpu.VMEM((NCH16, C), jnp.int32),
        pltpu.VMEM((NCH16, C), jnp.int32),
        pltpu.VMEM((C, DH), jnp.float32),
        pltpu.VMEM((C, DH), jnp.float32),
        pltpu.VMEM((16, DH), jnp.float32),
        pltpu.VMEM_SHARED((NPAD, DH), jnp.float32),
        pltpu.SemaphoreType.DMA,
        pltpu.SemaphoreType.DMA,
    ],
    compiler_params=pltpu.CompilerParams(use_tc_tiling_on_sc=False),
)(_edge_body)


BLK = 1024


def _prep_body(d0_ref, d1_ref, x_ref, a_ref, xp2_ref):
    i = pl.program_id(0)
    deg = d0_ref[...] + d1_ref[...]
    rows = lax.broadcasted_iota(jnp.int32, (BLK, 1), 0) + i * BLK
    a = jnp.where((deg > 0) & (rows < N),
                  lax.rsqrt(jnp.maximum(deg, 1e-30)), 0.0)
    a_ref[...] = a
    xp = a * x_ref[...]
    xp2_ref[0] = xp[:, :DH]
    xp2_ref[1] = xp[:, DH:]


_prep_kernel = pl.pallas_call(
    _prep_body,
    grid=(NPAD // BLK,),
    in_specs=[
        pl.BlockSpec((BLK, 1), lambda i: (i, 0)),
        pl.BlockSpec((BLK, 1), lambda i: (i, 0)),
        pl.BlockSpec((BLK, D), lambda i: (i, 0)),
    ],
    out_specs=[
        pl.BlockSpec((BLK, 1), lambda i: (i, 0)),
        pl.BlockSpec((2, BLK, DH), lambda i: (0, i, 0)),
    ],
    out_shape=[
        jax.ShapeDtypeStruct((NPAD, 1), jnp.float32),
        jax.ShapeDtypeStruct((2, NPAD, DH), jnp.float32),
    ],
)


def _comb_body(y_ref, a_ref, s_ref, so_ref, xp2_ref):
    a = a_ref[...]
    xa = a * y_ref[0]
    xb = a * y_ref[1]
    so_ref[...] = s_ref[...] + jnp.concatenate([xa, xb], axis=1)
    xp2_ref[0] = a * xa
    xp2_ref[1] = a * xb


_comb_kernel = pl.pallas_call(
    _comb_body,
    grid=(NPAD // BLK,),
    in_specs=[
        pl.BlockSpec((2, BLK, DH), lambda i: (0, i, 0)),
        pl.BlockSpec((BLK, 1), lambda i: (i, 0)),
        pl.BlockSpec((BLK, D), lambda i: (i, 0)),
    ],
    out_specs=[
        pl.BlockSpec((BLK, D), lambda i: (i, 0)),
        pl.BlockSpec((2, BLK, DH), lambda i: (0, i, 0)),
    ],
    out_shape=[
        jax.ShapeDtypeStruct((NPAD, D), jnp.float32),
        jax.ShapeDtypeStruct((2, NPAD, DH), jnp.float32),
    ],
)


def _final_body(y_ref, a_ref, s_ref, o_ref):
    a = a_ref[...]
    x = jnp.concatenate([a * y_ref[0], a * y_ref[1]], axis=1)
    o_ref[...] = (s_ref[...] + x) * jnp.float32(1.0 / (LAYERS + 1))


_final_kernel = pl.pallas_call(
    _final_body,
    grid=(NPAD // BLK,),
    in_specs=[
        pl.BlockSpec((2, BLK, DH), lambda i: (0, i, 0)),
        pl.BlockSpec((BLK, 1), lambda i: (i, 0)),
        pl.BlockSpec((BLK, D), lambda i: (i, 0)),
    ],
    out_specs=[pl.BlockSpec((BLK, D), lambda i: (i, 0))],
    out_shape=[jax.ShapeDtypeStruct((NPAD, D), jnp.float32)],
)


def kernel(edge_index, emb_weight):
    src = edge_index[0].astype(jnp.int32)
    dst = edge_index[1].astype(jnp.int32)
    pad32 = jnp.full((EPAD32 - E,), N, jnp.int32)
    dst_p32 = jnp.concatenate([dst, pad32]).reshape(NW, NCH32, C)
    pad16 = jnp.full((EPAD16 - E,), N, jnp.int32)
    src_p16 = jnp.concatenate([src, pad16]).reshape(NS, NCH16, C)
    dst_p16 = jnp.concatenate([dst, pad16]).reshape(NS, NCH16, C)
    x0 = jnp.pad(emb_weight, ((0, NPAD - N), (0, 0)))

    deg2 = _deg_kernel(dst_p32)
    a, xp2 = _prep_kernel(deg2[0, :, None], deg2[1, :, None], x0)
    s = x0
    for layer in range(LAYERS):
        y = _edge_kernel(src_p16, dst_p16, xp2)
        if layer + 1 < LAYERS:
            s, xp2 = _comb_kernel(y, a, s)
        else:
            (out,) = _final_kernel(y, a, s)
    return (emb_weight, out[:N])


# speedup vs baseline: 15.6667x; 1.1338x over previous
import functools

import jax
import jax.numpy as jnp
from jax import lax
from jax.experimental import pallas as pl
from jax.experimental.pallas import tpu as pltpu
from jax.experimental.pallas import tpu_sc as plsc

N = 10000
D = 128
DH = 64
E = 320000
LAYERS = 3
NC, NS = 2, 16
NW = NC * NS
NPAD = 10240
C = 128
RPS = NPAD // NS

EW32 = 10112
NCH32 = EW32 // C
EPAD32 = NW * EW32

EW16 = 20096
NCH16 = EW16 // C
EPAD16 = NS * EW16


def _deg_body(dst_hbm, deg_out, dst_v, ones_v, zvec, acc_sh, zsem, ssem):
    c = lax.axis_index("c")
    s = lax.axis_index("s")
    w = s * NC + c
    for i in range(RPS // 16):
        zvec[pl.ds(i * 16, 16)] = jnp.zeros((16,), jnp.float32)
    for i in range(C // 16):
        ones_v[pl.ds(i * 16, 16)] = jnp.ones((16,), jnp.float32)
    pltpu.sync_copy(dst_hbm.at[w], dst_v)
    pltpu.sync_copy(zvec, acc_sh.at[pl.ds(s * RPS, RPS)])
    plsc.subcore_barrier()
    K = 8
    for g in range(0, NCH32, K):
        n = min(K, NCH32 - g)
        cps = [
            pltpu.async_copy(ones_v, acc_sh.at[dst_v.at[g + t]], ssem, add=True)
            for t in range(n)
        ]
        for cp in cps:
            cp.wait()
    plsc.subcore_barrier()
    pltpu.sync_copy(acc_sh.at[pl.ds(s * RPS, RPS)],
                    deg_out.at[c, pl.ds(s * RPS, RPS)])


_deg_kernel = functools.partial(
    pl.kernel,
    out_type=jax.ShapeDtypeStruct((NC, NPAD), jnp.float32),
    mesh=plsc.VectorSubcoreMesh(core_axis_name="c", subcore_axis_name="s"),
    scratch_types=[
        pltpu.VMEM((NCH32, C), jnp.int32),
        pltpu.VMEM((C,), jnp.float32),
        pltpu.VMEM((RPS,), jnp.float32),
        pltpu.VMEM_SHARED((NPAD,), jnp.float32),
        pltpu.SemaphoreType.DMA,
        pltpu.SemaphoreType.DMA,
    ],
)(_deg_body)


def _edge_body(src_hbm, dst_hbm, xp2_hbm, y_out,
               src_v, dst_v, rows0, rows1, rows2, rows3, zrow, acc_sh,
               gsem0, gsem1, gsem2, gsem3):
    c = lax.axis_index("c")
    s = lax.axis_index("s")
    for r in range(16):
        for q in range(DH // 16):
            zrow[r, pl.ds(q * 16, 16)] = jnp.zeros((16,), jnp.float32)
    pltpu.sync_copy(src_hbm.at[s], src_v)
    pltpu.sync_copy(dst_hbm.at[s], dst_v)
    base = s * RPS
    zcps = [
        pltpu.async_copy(zrow, acc_sh.at[pl.ds(base + i * 16, 16)], gsem0)
        for i in range(RPS // 16)
    ]
    for cp in zcps:
        cp.wait()
    plsc.subcore_barrier()
    xp_h = xp2_hbm.at[c]
    bufs = (rows0, rows1, rows2, rows3)
    sems = (gsem0, gsem1, gsem2, gsem3)
    NB = len(bufs)
    cps = [None] * NB
    for j in range(NB - 1):
        cps[j] = pltpu.async_copy(xp_h.at[src_v.at[j]], bufs[j], sems[j])
    for j in range(NCH16):
        b = j % NB
        if j + NB - 1 < NCH16:
            nb = (j + NB - 1) % NB
            cps[nb] = pltpu.async_copy(
                xp_h.at[src_v.at[j + NB - 1]], bufs[nb], sems[nb])
        cps[b].wait()
        pltpu.sync_copy(bufs[b], acc_sh.at[dst_v.at[j]], add=True)
    plsc.subcore_barrier()
    pltpu.sync_copy(acc_sh.at[pl.ds(base, RPS)],
                    y_out.at[c, pl.ds(base, RPS)])


_edge_kernel = functools.partial(
    pl.kernel,
    out_type=jax.ShapeDtypeStruct((NC, NPAD, DH), jnp.float32),
    mesh=plsc.VectorSubcoreMesh(core_axis_name="c", subcore_axis_name="s"),
    scratch_types=[
        pltpu.VMEM((NCH16, C), jnp.int32),
        pltpu.VMEM((NCH16, C), jnp.int32),
        pltpu.VMEM((C, DH), jnp.float32),
        pltpu.VMEM((C, DH), jnp.float32),
        pltpu.VMEM((C, DH), jnp.float32),
        pltpu.VMEM((C, DH), jnp.float32),
        pltpu.VMEM((16, DH), jnp.float32),
        pltpu.VMEM_SHARED((NPAD, DH), jnp.float32),
        pltpu.SemaphoreType.DMA,
        pltpu.SemaphoreType.DMA,
        pltpu.SemaphoreType.DMA,
        pltpu.SemaphoreType.DMA,
    ],
    compiler_params=pltpu.CompilerParams(use_tc_tiling_on_sc=False),
)(_edge_body)


BLK = 1024


def _prep_body(d0_ref, d1_ref, x_ref, a_ref, xp2_ref):
    i = pl.program_id(0)
    deg = d0_ref[...] + d1_ref[...]
    rows = lax.broadcasted_iota(jnp.int32, (BLK, 1), 0) + i * BLK
    a = jnp.where((deg > 0) & (rows < N),
                  lax.rsqrt(jnp.maximum(deg, 1e-30)), 0.0)
    a_ref[...] = a
    xp = a * x_ref[...]
    xp2_ref[0] = xp[:, :DH]
    xp2_ref[1] = xp[:, DH:]


_prep_kernel = pl.pallas_call(
    _prep_body,
    grid=(NPAD // BLK,),
    in_specs=[
        pl.BlockSpec((BLK, 1), lambda i: (i, 0)),
        pl.BlockSpec((BLK, 1), lambda i: (i, 0)),
        pl.BlockSpec((BLK, D), lambda i: (i, 0)),
    ],
    out_specs=[
        pl.BlockSpec((BLK, 1), lambda i: (i, 0)),
        pl.BlockSpec((2, BLK, DH), lambda i: (0, i, 0)),
    ],
    out_shape=[
        jax.ShapeDtypeStruct((NPAD, 1), jnp.float32),
        jax.ShapeDtypeStruct((2, NPAD, DH), jnp.float32),
    ],
)


def _comb_body(y_ref, a_ref, s_ref, so_ref, xp2_ref):
    a = a_ref[...]
    xa = a * y_ref[0]
    xb = a * y_ref[1]
    so_ref[...] = s_ref[...] + jnp.concatenate([xa, xb], axis=1)
    xp2_ref[0] = a * xa
    xp2_ref[1] = a * xb


_comb_kernel = pl.pallas_call(
    _comb_body,
    grid=(NPAD // BLK,),
    in_specs=[
        pl.BlockSpec((2, BLK, DH), lambda i: (0, i, 0)),
        pl.BlockSpec((BLK, 1), lambda i: (i, 0)),
        pl.BlockSpec((BLK, D), lambda i: (i, 0)),
    ],
    out_specs=[
        pl.BlockSpec((BLK, D), lambda i: (i, 0)),
        pl.BlockSpec((2, BLK, DH), lambda i: (0, i, 0)),
    ],
    out_shape=[
        jax.ShapeDtypeStruct((NPAD, D), jnp.float32),
        jax.ShapeDtypeStruct((2, NPAD, DH), jnp.float32),
    ],
)


def _final_body(y_ref, a_ref, s_ref, o_ref):
    a = a_ref[...]
    x = jnp.concatenate([a * y_ref[0], a * y_ref[1]], axis=1)
    o_ref[...] = (s_ref[...] + x) * jnp.float32(1.0 / (LAYERS + 1))


_final_kernel = pl.pallas_call(
    _final_body,
    grid=(NPAD // BLK,),
    in_specs=[
        pl.BlockSpec((2, BLK, DH), lambda i: (0, i, 0)),
        pl.BlockSpec((BLK, 1), lambda i: (i, 0)),
        pl.BlockSpec((BLK, D), lambda i: (i, 0)),
    ],
    out_specs=[pl.BlockSpec((BLK, D), lambda i: (i, 0))],
    out_shape=[jax.ShapeDtypeStruct((NPAD, D), jnp.float32)],
)


def kernel(edge_index, emb_weight):
    src = edge_index[0].astype(jnp.int32)
    dst = edge_index[1].astype(jnp.int32)
    pad32 = jnp.full((EPAD32 - E,), N, jnp.int32)
    dst_p32 = jnp.concatenate([dst, pad32]).reshape(NW, NCH32, C)
    pad16 = jnp.full((EPAD16 - E,), N, jnp.int32)
    src_p16 = jnp.concatenate([src, pad16]).reshape(NS, NCH16, C)
    dst_p16 = jnp.concatenate([dst, pad16]).reshape(NS, NCH16, C)
    x0 = jnp.pad(emb_weight, ((0, NPAD - N), (0, 0)))

    deg2 = _deg_kernel(dst_p32)
    a, xp2 = _prep_kernel(deg2[0, :, None], deg2[1, :, None], x0)
    s = x0
    for layer in range(LAYERS):
        y = _edge_kernel(src_p16, dst_p16, xp2)
        if layer + 1 < LAYERS:
            s, xp2 = _comb_kernel(y, a, s)
        else:
            (out,) = _final_kernel(y, a, s)
    return (emb_weight, out[:N])
